# Initial kernel scaffold; baseline (speedup 1.0000x reference)
import jax
import jax.numpy as jnp
from jax.experimental import pallas as pl


def kernel(outputs, targets):
    raise NotImplementedError("write your pallas kernel here")


# speedup vs baseline: 13.0623x; 13.0623x over previous
import functools

import jax
import jax.numpy as jnp
from jax import lax
from jax.experimental import pallas as pl
from jax.experimental.pallas import tpu as pltpu
from jax.experimental.pallas import tpu_sc as plsc

_NC = 2
_NS = 16
_NW = _NC * _NS
_L = 16
_CHUNK = 4096

_B1 = 2048
_B2 = 2048
_B3 = 512
_I32MAX = 2147483647


def _mesh():
    return plsc.VectorSubcoreMesh(core_axis_name="c", subcore_axis_name="s")


def _wid():
    return lax.axis_index("s") * _NC + lax.axis_index("c")


def _loss_bits(obuf, tbuf, j):
    o16 = obuf[pl.ds(j * _L, _L)]
    t16 = tbuf[pl.ds(j * _L, _L)]
    d = o16 - t16
    l = d * d
    return l, lax.bitcast_convert_type(l, jnp.int32)


def _zero_hist(hist, words):
    z = jnp.zeros((_L,), jnp.int32)

    def body(b, _):
        hist[pl.ds(b * _L, _L)] = z
        return _

    lax.fori_loop(0, words // _L, body, None)


@functools.lru_cache(maxsize=None)
def _build(n):
    nw = n // _NW
    chunks = nw // _CHUNK
    assert nw % _CHUNK == 0

    @functools.partial(
        pl.kernel,
        out_type=jax.ShapeDtypeStruct((_NW, _B1 * _L), jnp.int32),
        mesh=_mesh(),
        compiler_params=pltpu.CompilerParams(needs_layout_passes=False),
        scratch_types=[
            pltpu.VMEM((_CHUNK,), jnp.float32),
            pltpu.VMEM((_CHUNK,), jnp.float32),
            pltpu.VMEM((_B1 * _L,), jnp.int32),
        ],
    )
    def pass1(o_hbm, t_hbm, out_hbm, obuf, tbuf, hist):
        lane = lax.iota(jnp.int32, _L)
        ones = jnp.ones((_L,), jnp.int32)
        _zero_hist(hist, _B1 * _L)
        base_w = _wid() * nw

        def chunk(i, _):
            base = pl.multiple_of(base_w + i * _CHUNK, _CHUNK)
            pltpu.sync_copy(o_hbm.at[pl.ds(base, _CHUNK)], obuf)
            pltpu.sync_copy(t_hbm.at[pl.ds(base, _CHUNK)], tbuf)

            def body(j, carry):
                lv, u = _loss_bits(obuf, tbuf, j)
                addr = ((u >> 20) << 4) + lane
                plsc.addupdate_scatter(hist, [addr], ones)
                return carry

            lax.fori_loop(0, _CHUNK // _L, body, None)
            return _

        lax.fori_loop(0, chunks, chunk, None)
        pltpu.sync_copy(hist, out_hbm.at[_wid()])

    @functools.partial(
        pl.kernel,
        out_type=jax.ShapeDtypeStruct((_NW, 2 * _B2 * _L), jnp.int32),
        mesh=_mesh(),
        compiler_params=pltpu.CompilerParams(needs_layout_passes=False),
        scratch_types=[
            pltpu.VMEM((_CHUNK,), jnp.float32),
            pltpu.VMEM((_CHUNK,), jnp.float32),
            pltpu.VMEM((2 * _B2 * _L,), jnp.int32),
            pltpu.VMEM((2 * _L,), jnp.int32),
        ],
    )
    def pass2(o_hbm, t_hbm, pfx_hbm, out_hbm, obuf, tbuf, hist, pbuf):
        lane = lax.iota(jnp.int32, _L)
        ones = jnp.ones((_L,), jnp.int32)
        _zero_hist(hist, 2 * _B2 * _L)
        pltpu.sync_copy(pfx_hbm, pbuf)
        p0 = pbuf[pl.ds(0, _L)]
        p1 = pbuf[pl.ds(_L, _L)]
        base_w = _wid() * nw

        def chunk(i, _):
            base = pl.multiple_of(base_w + i * _CHUNK, _CHUNK)
            pltpu.sync_copy(o_hbm.at[pl.ds(base, _CHUNK)], obuf)
            pltpu.sync_copy(t_hbm.at[pl.ds(base, _CHUNK)], tbuf)

            def body(j, carry):
                lv, u = _loss_bits(obuf, tbuf, j)
                top = u >> 20
                addr = (((u >> 9) & (_B2 - 1)) << 4) + lane
                plsc.addupdate_scatter(hist, [addr], ones, mask=top == p0)
                plsc.addupdate_scatter(
                    hist, [addr + _B2 * _L], ones, mask=top == p1)
                return carry

            lax.fori_loop(0, _CHUNK // _L, body, None)
            return _

        lax.fori_loop(0, chunks, chunk, None)
        pltpu.sync_copy(hist, out_hbm.at[_wid()])

    @functools.partial(
        pl.kernel,
        out_type=(
            jax.ShapeDtypeStruct((_NW, 2 * _B3 * _L), jnp.int32),
            jax.ShapeDtypeStruct((_NW, 2 * _L), jnp.int32),
        ),
        mesh=_mesh(),
        compiler_params=pltpu.CompilerParams(needs_layout_passes=False),
        scratch_types=[
            pltpu.VMEM((_CHUNK,), jnp.float32),
            pltpu.VMEM((_CHUNK,), jnp.float32),
            pltpu.VMEM((2 * _B3 * _L,), jnp.int32),
            pltpu.VMEM((2 * _L,), jnp.int32),
            pltpu.VMEM((2 * _L,), jnp.int32),
            pltpu.VMEM((2 * _L,), jnp.int32),
        ],
    )
    def pass3(o_hbm, t_hbm, pfx_hbm, thr_hbm, hist_hbm, min_hbm,
              obuf, tbuf, hist, pbuf, tbits, mstage):
        lane = lax.iota(jnp.int32, _L)
        ones = jnp.ones((_L,), jnp.int32)
        imax = jnp.full((_L,), _I32MAX, jnp.int32)
        _zero_hist(hist, 2 * _B3 * _L)
        pltpu.sync_copy(pfx_hbm, pbuf)
        pltpu.sync_copy(thr_hbm, tbits)
        p0 = pbuf[pl.ds(0, _L)]
        p1 = pbuf[pl.ds(_L, _L)]
        t0 = tbits[pl.ds(0, _L)]
        t1 = tbits[pl.ds(_L, _L)]
        base_w = _wid() * nw

        def chunk(i, carry):
            m0, m1 = carry
            base = pl.multiple_of(base_w + i * _CHUNK, _CHUNK)
            pltpu.sync_copy(o_hbm.at[pl.ds(base, _CHUNK)], obuf)
            pltpu.sync_copy(t_hbm.at[pl.ds(base, _CHUNK)], tbuf)

            def body(j, c):
                m0, m1 = c
                _, u = _loss_bits(obuf, tbuf, j)
                mid = u >> 9
                addr = ((u & (_B3 - 1)) << 4) + lane
                plsc.addupdate_scatter(hist, [addr], ones, mask=mid == p0)
                plsc.addupdate_scatter(
                    hist, [addr + _B3 * _L], ones, mask=mid == p1)
                m0 = jnp.minimum(m0, jnp.where(u >= t0, u, imax))
                m1 = jnp.minimum(m1, jnp.where(u >= t1, u, imax))
                return m0, m1

            return lax.fori_loop(0, _CHUNK // _L, body, (m0, m1))

        m0, m1 = lax.fori_loop(0, chunks, chunk, (imax, imax))
        mstage[pl.ds(0, _L)] = m0
        mstage[pl.ds(_L, _L)] = m1
        pltpu.sync_copy(hist, hist_hbm.at[_wid()])
        pltpu.sync_copy(mstage, min_hbm.at[_wid()])

    @functools.partial(
        pl.kernel,
        out_type=(
            jax.ShapeDtypeStruct((_NW, _L), jnp.float32),
            jax.ShapeDtypeStruct((_NW, _L), jnp.int32),
        ),
        mesh=_mesh(),
        compiler_params=pltpu.CompilerParams(needs_layout_passes=False),
        scratch_types=[
            pltpu.VMEM((_CHUNK,), jnp.float32),
            pltpu.VMEM((_CHUNK,), jnp.float32),
            pltpu.VMEM((_L,), jnp.float32),
            pltpu.VMEM((_L,), jnp.float32),
            pltpu.VMEM((_L,), jnp.int32),
        ],
    )
    def pass4(o_hbm, t_hbm, thr_hbm, sum_hbm, cnt_hbm,
              obuf, tbuf, thrv, sstage, cstage):
        zf = jnp.zeros((_L,), jnp.float32)
        zi = jnp.zeros((_L,), jnp.int32)
        onei = jnp.ones((_L,), jnp.int32)
        pltpu.sync_copy(thr_hbm, thrv)
        thr = thrv[pl.ds(0, _L)]
        base_w = _wid() * nw

        def chunk(i, carry):
            s, c = carry
            base = pl.multiple_of(base_w + i * _CHUNK, _CHUNK)
            pltpu.sync_copy(o_hbm.at[pl.ds(base, _CHUNK)], obuf)
            pltpu.sync_copy(t_hbm.at[pl.ds(base, _CHUNK)], tbuf)

            def body(j, cc):
                s, c = cc
                l, _ = _loss_bits(obuf, tbuf, j)
                keep = l < thr
                s = s + jnp.where(keep, l, 0.0)
                c = c + jnp.where(keep, onei, zi)
                return s, c

            return lax.fori_loop(0, _CHUNK // _L, body, (s, c))

        s, c = lax.fori_loop(0, chunks, chunk, (zf, zi))
        sstage[pl.ds(0, _L)] = s
        cstage[pl.ds(0, _L)] = c
        pltpu.sync_copy(sstage, sum_hbm.at[_wid()])
        pltpu.sync_copy(cstage, cnt_hbm.at[_wid()])

    return pass1, pass2, pass3, pass4


def _rank_split(cum, r):
    b = jnp.searchsorted(cum, r, side="right")
    below = jnp.where(b > 0, cum[jnp.maximum(b - 1, 0)], 0)
    return b.astype(jnp.int32), (r - below).astype(jnp.int32)


def kernel(outputs, targets):
    o = outputs.reshape(-1)
    t = targets.reshape(-1)
    n = o.shape[0]
    p1, p2, p3, p4 = _build(n)

    k_lo = [(n - 1) // 4, (3 * (n - 1)) // 4]
    frac = jnp.array(
        [((n - 1) % 4) / 4.0, ((3 * (n - 1)) % 4) / 4.0], jnp.float32)

    h1 = p1(o, t).reshape(_NW, _B1, _L).sum(axis=(0, 2))
    c1 = jnp.cumsum(h1)
    b1 = []
    r1 = []
    for k in k_lo:
        b, r = _rank_split(c1, jnp.int32(k))
        b1.append(b)
        r1.append(r)
    b1 = jnp.stack(b1)
    r1 = jnp.stack(r1)

    pfx1 = jnp.broadcast_to(b1[:, None], (2, _L)).reshape(-1)
    h2 = p2(o, t, pfx1).reshape(_NW, 2, _B2, _L).sum(axis=(0, 3))
    c2 = jnp.cumsum(h2, axis=1)
    b2, r2 = jax.vmap(_rank_split)(c2, r1)

    pfx22 = (b1 << 11) | b2
    thr_bits = (pfx22 + 1) << 9
    pfx22_b = jnp.broadcast_to(pfx22[:, None], (2, _L)).reshape(-1)
    thr_b = jnp.broadcast_to(thr_bits[:, None], (2, _L)).reshape(-1)
    h3_raw, min_raw = p3(o, t, pfx22_b, thr_b)
    h3 = h3_raw.reshape(_NW, 2, _B3, _L).sum(axis=(0, 3))
    min_above = min_raw.reshape(_NW, 2, _L).min(axis=(0, 2))
    c3 = jnp.cumsum(h3, axis=1)
    b3, _ = jax.vmap(_rank_split)(c3, r2)

    bits_lo = (pfx22 << 9) | b3
    cum_at = jnp.take_along_axis(c3, b3[:, None].astype(jnp.int32), axis=1)[:, 0]
    in_bucket = (r2 + 1) < c3[:, _B3 - 1]
    b3h = jax.vmap(
        lambda c, r: jnp.searchsorted(c, r, side="right"))(c3, r2 + 1)
    bits_hi = jnp.where(
        (r2 + 1) < cum_at,
        bits_lo,
        jnp.where(in_bucket, (pfx22 << 9) | b3h.astype(jnp.int32), min_above),
    )

    v_lo = lax.bitcast_convert_type(bits_lo, jnp.float32)
    v_hi = lax.bitcast_convert_type(bits_hi, jnp.float32)
    q = v_lo + frac * (v_hi - v_lo)
    thresh = 2.5 * q[1] - 1.5 * q[0]

    thr_bcast = jnp.broadcast_to(thresh, (_L,)).astype(jnp.float32)
    sums, cnts = p4(o, t, thr_bcast)
    total = jnp.sum(sums)
    count = jnp.sum(cnts)
    return total / count.astype(jnp.float32)


# speedup vs baseline: 46.7300x; 3.5775x over previous
import functools

import jax
import jax.numpy as jnp
from jax import lax
from jax.experimental import pallas as pl
from jax.experimental.pallas import tpu as pltpu
from jax.experimental.pallas import tpu_sc as plsc

_NC = 2
_NS = 16
_NW = _NC * _NS
_L = 16
_CHUNK = 4096
_UNROLL = 4

_S1 = 15
_B1 = 1 << (31 - _S1)


def _mesh():
    return plsc.VectorSubcoreMesh(core_axis_name="c", subcore_axis_name="s")


def _params():
    return pltpu.CompilerParams(needs_layout_passes=False)


def _wid():
    return lax.axis_index("s") * _NC + lax.axis_index("c")


def _loss_bits(ob, tb, j):
    o16 = ob[pl.ds(j * _L, _L)]
    t16 = tb[pl.ds(j * _L, _L)]
    d = o16 - t16
    l = d * d
    return l, lax.bitcast_convert_type(l, jnp.int32)


def _zero_hist(hist, words):
    z = jnp.zeros((_L,), jnp.int32)

    def body(b, carry):
        hist[pl.ds(b * _L, _L)] = z
        return carry

    lax.fori_loop(0, words // _L, body, None)


def _stream(o_hbm, t_hbm, bufs, sem0, sem1, base_w, chunks,
            body_fn, carry_init):
    (ob0, tb0), (ob1, tb1) = bufs
    sems = (sem0, sem1)
    obufs = (ob0, ob1)
    tbufs = (tb0, tb1)

    def start(i, p):
        base = pl.multiple_of(base_w + i * _CHUNK, _CHUNK)
        pltpu.async_copy(o_hbm.at[pl.ds(base, _CHUNK)], obufs[p], sems[p])
        pltpu.async_copy(t_hbm.at[pl.ds(base, _CHUNK)], tbufs[p], sems[p])

    def wait(p):
        pltpu.make_async_copy(
            o_hbm.at[pl.ds(0, _CHUNK)], obufs[p], sems[p]).wait()
        pltpu.make_async_copy(
            t_hbm.at[pl.ds(0, _CHUNK)], tbufs[p], sems[p]).wait()

    start(0, 0)

    def pair(ip, carry):
        i = ip * 2
        start(i + 1, 1)
        wait(0)
        carry = body_fn(obufs[0], tbufs[0], carry)

        @pl.when(i + 2 < chunks)
        def _():
            start(i + 2, 0)

        wait(1)
        carry = body_fn(obufs[1], tbufs[1], carry)
        return carry

    return lax.fori_loop(0, chunks // 2, pair, carry_init)


def _inner(body1, carry_init):
    def body(jj, carry):
        for v in range(_UNROLL):
            carry = body1(jj * _UNROLL + v, carry)
        return carry

    return lax.fori_loop(0, _CHUNK // _L // _UNROLL, body, carry_init)


@functools.lru_cache(maxsize=None)
def _build(n):
    nw = n // _NW
    chunks = nw // _CHUNK
    assert nw % _CHUNK == 0 and chunks % 2 == 0

    @functools.partial(
        pl.kernel,
        out_type=jax.ShapeDtypeStruct((_NW, _B1), jnp.int32),
        mesh=_mesh(),
        compiler_params=_params(),
        scratch_types=[
            pltpu.VMEM((_CHUNK,), jnp.float32),
            pltpu.VMEM((_CHUNK,), jnp.float32),
            pltpu.VMEM((_CHUNK,), jnp.float32),
            pltpu.VMEM((_CHUNK,), jnp.float32),
            pltpu.VMEM((_B1,), jnp.int32),
            pltpu.SemaphoreType.DMA,
            pltpu.SemaphoreType.DMA,
        ],
    )
    def pass1(o_hbm, t_hbm, out_hbm, ob0, tb0, ob1, tb1, hist, sem0, sem1):
        ones = jnp.ones((_L,), jnp.int32)
        _zero_hist(hist, _B1)

        def chunk_body(ob, tb, carry):
            def body1(j, c):
                lv, u = _loss_bits(ob, tb, j)
                plsc.addupdate_scatter(hist, [u >> _S1], ones)
                return c

            return _inner(body1, carry)

        _stream(o_hbm, t_hbm, ((ob0, tb0), (ob1, tb1)), sem0, sem1,
                _wid() * nw, chunks, chunk_body, 0)
        pltpu.sync_copy(hist, out_hbm.at[_wid()])

    @functools.partial(
        pl.kernel,
        out_type=(
            jax.ShapeDtypeStruct((_NW, _L), jnp.float32),
            jax.ShapeDtypeStruct((_NW, _L), jnp.int32),
        ),
        mesh=_mesh(),
        compiler_params=_params(),
        scratch_types=[
            pltpu.VMEM((_CHUNK,), jnp.float32),
            pltpu.VMEM((_CHUNK,), jnp.float32),
            pltpu.VMEM((_CHUNK,), jnp.float32),
            pltpu.VMEM((_CHUNK,), jnp.float32),
            pltpu.VMEM((_L,), jnp.float32),
            pltpu.VMEM((_L,), jnp.float32),
            pltpu.VMEM((_L,), jnp.int32),
            pltpu.SemaphoreType.DMA,
            pltpu.SemaphoreType.DMA,
        ],
    )
    def pass2(o_hbm, t_hbm, thr_hbm, sum_hbm, cnt_hbm,
              ob0, tb0, ob1, tb1, thrv, sstage, cstage, sem0, sem1):
        zf = jnp.zeros((_L,), jnp.float32)
        zi = jnp.zeros((_L,), jnp.int32)
        onei = jnp.ones((_L,), jnp.int32)
        pltpu.sync_copy(thr_hbm, thrv)
        thr = thrv[pl.ds(0, _L)]

        def chunk_body(ob, tb, carry):
            def body1(j, c):
                s, cnt = c
                lv, u = _loss_bits(ob, tb, j)
                keep = lv < thr
                s = s + jnp.where(keep, lv, 0.0)
                cnt = cnt + jnp.where(keep, onei, zi)
                return s, cnt

            return _inner(body1, carry)

        s, cnt = _stream(o_hbm, t_hbm, ((ob0, tb0), (ob1, tb1)), sem0, sem1,
                         _wid() * nw, chunks, chunk_body, (zf, zi))
        sstage[pl.ds(0, _L)] = s
        cstage[pl.ds(0, _L)] = cnt
        pltpu.sync_copy(sstage, sum_hbm.at[_wid()])
        pltpu.sync_copy(cstage, cnt_hbm.at[_wid()])

    return pass1, pass2


def _rank_split(cum, r):
    b = jnp.searchsorted(cum, r, side="right")
    below = jnp.where(b > 0, cum[jnp.maximum(b - 1, 0)], 0)
    return b.astype(jnp.int32), (r - below).astype(jnp.int32)


def kernel(outputs, targets):
    o = outputs.reshape(-1)
    t = targets.reshape(-1)
    n = o.shape[0]
    p1, p2 = _build(n)

    k_lo = [(n - 1) // 4, (3 * (n - 1)) // 4]
    frac = jnp.array(
        [((n - 1) % 4) / 4.0, ((3 * (n - 1)) % 4) / 4.0], jnp.float32)

    h1 = p1(o, t).sum(axis=0)
    c1 = jnp.cumsum(h1)
    b1 = []
    r1 = []
    for k in k_lo:
        b, r = _rank_split(c1, jnp.int32(k))
        b1.append(b)
        r1.append(r)
    b1 = jnp.stack(b1)
    r1 = jnp.stack(r1)

    cnt_b = h1[b1]
    v_start = lax.bitcast_convert_type(b1 << _S1, jnp.float32)
    v_end = lax.bitcast_convert_type((b1 + 1) << _S1, jnp.float32)
    pos = (r1.astype(jnp.float32) + frac + 0.5) / cnt_b.astype(jnp.float32)
    q = v_start + pos * (v_end - v_start)
    thresh = 2.5 * q[1] - 1.5 * q[0]

    thr_bcast = jnp.broadcast_to(thresh, (_L,)).astype(jnp.float32)
    sums, cnts = p2(o, t, thr_bcast)
    total = jnp.sum(sums)
    count = jnp.sum(cnts)
    return total / count.astype(jnp.float32)


# speedup vs baseline: 50.1641x; 1.0735x over previous
import functools

import jax
import jax.numpy as jnp
from jax import lax
from jax.experimental import pallas as pl
from jax.experimental.pallas import tpu as pltpu
from jax.experimental.pallas import tpu_sc as plsc

_NC = 2
_NS = 16
_NW = _NC * _NS
_L = 16
_CHUNK = 4096
_UNROLL = 4

_S1 = 15
_B1 = 1 << (31 - _S1)


def _mesh():
    return plsc.VectorSubcoreMesh(core_axis_name="c", subcore_axis_name="s")


def _params():
    return pltpu.CompilerParams(needs_layout_passes=False)


def _wid():
    return lax.axis_index("s") * _NC + lax.axis_index("c")


def _loss_bits(ob, tb, j):
    o16 = ob[pl.ds(j * _L, _L)]
    t16 = tb[pl.ds(j * _L, _L)]
    d = o16 - t16
    l = d * d
    return l, lax.bitcast_convert_type(l, jnp.int32)


def _zero_hist(hist, words):
    z = jnp.zeros((_L,), jnp.int32)

    def body(b, carry):
        hist[pl.ds(b * _L, _L)] = z
        return carry

    lax.fori_loop(0, words // _L, body, None)


def _stream(o_hbm, t_hbm, bufs, sem0, sem1, base_w, chunks,
            body_fn, carry_init):
    (ob0, tb0), (ob1, tb1) = bufs
    sems = (sem0, sem1)
    obufs = (ob0, ob1)
    tbufs = (tb0, tb1)

    def start(i, p):
        base = pl.multiple_of(base_w + i * _CHUNK, _CHUNK)
        pltpu.async_copy(o_hbm.at[pl.ds(base, _CHUNK)], obufs[p], sems[p])
        pltpu.async_copy(t_hbm.at[pl.ds(base, _CHUNK)], tbufs[p], sems[p])

    def wait(p):
        pltpu.make_async_copy(
            o_hbm.at[pl.ds(0, _CHUNK)], obufs[p], sems[p]).wait()
        pltpu.make_async_copy(
            t_hbm.at[pl.ds(0, _CHUNK)], tbufs[p], sems[p]).wait()

    start(0, 0)

    def pair(ip, carry):
        i = ip * 2
        start(i + 1, 1)
        wait(0)
        carry = body_fn(obufs[0], tbufs[0], carry)

        @pl.when(i + 2 < chunks)
        def _():
            start(i + 2, 0)

        wait(1)
        carry = body_fn(obufs[1], tbufs[1], carry)
        return carry

    return lax.fori_loop(0, chunks // 2, pair, carry_init)


def _inner(body1, carry_init):
    def body(jj, carry):
        for v in range(_UNROLL):
            carry = body1(jj * _UNROLL + v, carry)
        return carry

    return lax.fori_loop(0, _CHUNK // _L // _UNROLL, body, carry_init)


@functools.lru_cache(maxsize=None)
def _build(n):
    nw = n // _NW
    chunks = nw // _CHUNK
    assert nw % _CHUNK == 0 and chunks % 2 == 0

    @functools.partial(
        pl.kernel,
        out_type=jax.ShapeDtypeStruct((_NW, _B1), jnp.int32),
        mesh=_mesh(),
        compiler_params=_params(),
        scratch_types=[
            pltpu.VMEM((_CHUNK,), jnp.float32),
            pltpu.VMEM((_CHUNK,), jnp.float32),
            pltpu.VMEM((_CHUNK,), jnp.float32),
            pltpu.VMEM((_CHUNK,), jnp.float32),
            pltpu.VMEM((_B1,), jnp.int32),
            pltpu.SemaphoreType.DMA,
            pltpu.SemaphoreType.DMA,
        ],
    )
    def pass1(o_hbm, t_hbm, out_hbm, ob0, tb0, ob1, tb1, hist, sem0, sem1):
        ones = jnp.ones((_L,), jnp.int32)
        _zero_hist(hist, _B1)

        def chunk_body(ob, tb, carry):
            def body1(j, c):
                lv, u = _loss_bits(ob, tb, j)
                plsc.addupdate_scatter(hist, [u >> _S1], ones)
                return c

            return _inner(body1, carry)

        _stream(o_hbm, t_hbm, ((ob0, tb0), (ob1, tb1)), sem0, sem1,
                _wid() * nw, chunks, chunk_body, 0)
        pltpu.sync_copy(hist, out_hbm.at[_wid()])

    return pass1


@functools.lru_cache(maxsize=None)
def _build_masked_sum(shape):
    b, rows, cols = shape
    br = 512
    grid = (b * rows) // br
    assert rows % br == 0

    def body(thr_ref, o_ref, t_ref, sum_ref, cnt_ref):
        i = pl.program_id(0)

        @pl.when(i == 0)
        def _():
            sum_ref[...] = jnp.zeros_like(sum_ref)
            cnt_ref[...] = jnp.zeros_like(cnt_ref)

        d = o_ref[...] - t_ref[...]
        l = d * d
        keep = l < thr_ref[0, 0]
        sum_ref[...] += jnp.sum(jnp.where(keep, l, 0.0))
        cnt_ref[...] += jnp.sum(keep.astype(jnp.int32))

    bpb = rows // br
    flat = lambda i: (i // bpb, i % bpb, 0)
    return pl.pallas_call(
        body,
        grid=(grid,),
        in_specs=[
            pl.BlockSpec(memory_space=pltpu.SMEM),
            pl.BlockSpec((1, br, cols), flat),
            pl.BlockSpec((1, br, cols), flat),
        ],
        out_specs=(
            pl.BlockSpec((8, 128), lambda i: (0, 0)),
            pl.BlockSpec((8, 128), lambda i: (0, 0)),
        ),
        out_shape=(
            jax.ShapeDtypeStruct((8, 128), jnp.float32),
            jax.ShapeDtypeStruct((8, 128), jnp.int32),
        ),
    )


def _rank_split(cum, r):
    b = jnp.searchsorted(cum, r, side="right")
    below = jnp.where(b > 0, cum[jnp.maximum(b - 1, 0)], 0)
    return b.astype(jnp.int32), (r - below).astype(jnp.int32)


def kernel(outputs, targets):
    o = outputs.reshape(-1)
    t = targets.reshape(-1)
    n = o.shape[0]
    p1 = _build(n)
    p2 = _build_masked_sum(outputs.shape)

    k_lo = [(n - 1) // 4, (3 * (n - 1)) // 4]
    frac = jnp.array(
        [((n - 1) % 4) / 4.0, ((3 * (n - 1)) % 4) / 4.0], jnp.float32)

    h1 = p1(o, t).sum(axis=0)
    c1 = jnp.cumsum(h1)
    b1 = []
    r1 = []
    for k in k_lo:
        b, r = _rank_split(c1, jnp.int32(k))
        b1.append(b)
        r1.append(r)
    b1 = jnp.stack(b1)
    r1 = jnp.stack(r1)

    cnt_b = h1[b1]
    v_start = lax.bitcast_convert_type(b1 << _S1, jnp.float32)
    v_end = lax.bitcast_convert_type((b1 + 1) << _S1, jnp.float32)
    pos = (r1.astype(jnp.float32) + frac + 0.5) / cnt_b.astype(jnp.float32)
    q = v_start + pos * (v_end - v_start)
    thresh = 2.5 * q[1] - 1.5 * q[0]

    thr_smem = thresh.reshape(1, 1).astype(jnp.float32)
    sums, cnts = p2(thr_smem, outputs, targets)
    total = sums[0, 0]
    count = cnts[0, 0]
    return total / count.astype(jnp.float32)


# speedup vs baseline: 64.2558x; 1.2809x over previous
import functools

import jax
import jax.numpy as jnp
from jax import lax
from jax.experimental import pallas as pl
from jax.experimental.pallas import tpu as pltpu
from jax.experimental.pallas import tpu_sc as plsc

_NC = 2
_NS = 16
_NW = _NC * _NS
_L = 16
_CHUNK = 4096
_UNROLL = 4

_S1 = 15
_B1 = 1 << (31 - _S1)


def _mesh():
    return plsc.VectorSubcoreMesh(core_axis_name="c", subcore_axis_name="s")


def _params():
    return pltpu.CompilerParams(needs_layout_passes=False)


def _wid():
    return lax.axis_index("s") * _NC + lax.axis_index("c")


def _zero_hist(hist, words):
    z = jnp.zeros((_L,), jnp.int32)

    def body(b, carry):
        hist[pl.ds(b * _L, _L)] = z
        return carry

    lax.fori_loop(0, words // _L, body, None)


def _stream(l_hbm, bufs, sem0, sem1, base_w, chunks, body_fn, carry_init):
    sems = (sem0, sem1)

    def start(i, p):
        base = pl.multiple_of(base_w + i * _CHUNK, _CHUNK)
        pltpu.async_copy(l_hbm.at[pl.ds(base, _CHUNK)], bufs[p], sems[p])

    def wait(p):
        pltpu.make_async_copy(
            l_hbm.at[pl.ds(0, _CHUNK)], bufs[p], sems[p]).wait()

    start(0, 0)

    def pair(ip, carry):
        i = ip * 2
        start(i + 1, 1)
        wait(0)
        carry = body_fn(bufs[0], carry)

        @pl.when(i + 2 < chunks)
        def _():
            start(i + 2, 0)

        wait(1)
        carry = body_fn(bufs[1], carry)
        return carry

    return lax.fori_loop(0, chunks // 2, pair, carry_init)


def _inner(body1, carry_init):
    def body(jj, carry):
        for v in range(_UNROLL):
            carry = body1(jj * _UNROLL + v, carry)
        return carry

    return lax.fori_loop(0, _CHUNK // _L // _UNROLL, body, carry_init)


@functools.lru_cache(maxsize=None)
def _build(n):
    nw = n // _NW
    chunks = nw // _CHUNK
    assert nw % _CHUNK == 0 and chunks % 2 == 0

    @functools.partial(
        pl.kernel,
        out_type=jax.ShapeDtypeStruct((_NW, _B1), jnp.int32),
        mesh=_mesh(),
        compiler_params=_params(),
        scratch_types=[
            pltpu.VMEM((_CHUNK,), jnp.float32),
            pltpu.VMEM((_CHUNK,), jnp.float32),
            pltpu.VMEM((_B1,), jnp.int32),
            pltpu.SemaphoreType.DMA,
            pltpu.SemaphoreType.DMA,
        ],
    )
    def pass1(l_hbm, out_hbm, lb0, lb1, hist, sem0, sem1):
        ones = jnp.ones((_L,), jnp.int32)
        _zero_hist(hist, _B1)

        def chunk_body(lb, carry):
            def body1(j, c):
                lv = lb[pl.ds(j * _L, _L)]
                u = lax.bitcast_convert_type(lv, jnp.int32)
                plsc.addupdate_scatter(hist, [u >> _S1], ones)
                return c

            return _inner(body1, carry)

        _stream(l_hbm, (lb0, lb1), sem0, sem1,
                _wid() * nw, chunks, chunk_body, 0)
        pltpu.sync_copy(hist, out_hbm.at[_wid()])

    return pass1


@functools.lru_cache(maxsize=None)
def _build_loss(shape):
    b, rows, cols = shape
    br = 512
    grid = (b * rows) // br
    bpb = rows // br
    flat3 = lambda i: (i // bpb, i % bpb, 0)

    def body(o_ref, t_ref, l_ref):
        d = o_ref[...] - t_ref[...]
        l_ref[...] = (d * d).reshape(-1)

    return pl.pallas_call(
        body,
        grid=(grid,),
        in_specs=[
            pl.BlockSpec((1, br, cols), flat3),
            pl.BlockSpec((1, br, cols), flat3),
        ],
        out_specs=pl.BlockSpec((br * cols,), lambda i: (i,)),
        out_shape=jax.ShapeDtypeStruct((b * rows * cols,), jnp.float32),
    )


@functools.lru_cache(maxsize=None)
def _build_masked_sum(shape):
    b, rows, cols = shape
    br = 512
    grid = (b * rows) // br
    assert rows % br == 0

    def body(thr_ref, o_ref, t_ref, sum_ref, cnt_ref):
        i = pl.program_id(0)

        @pl.when(i == 0)
        def _():
            sum_ref[...] = jnp.zeros_like(sum_ref)
            cnt_ref[...] = jnp.zeros_like(cnt_ref)

        d = o_ref[...] - t_ref[...]
        l = d * d
        keep = l < thr_ref[0, 0]
        sum_ref[...] += jnp.sum(jnp.where(keep, l, 0.0))
        cnt_ref[...] += jnp.sum(keep.astype(jnp.int32))

    bpb = rows // br
    flat = lambda i: (i // bpb, i % bpb, 0)
    return pl.pallas_call(
        body,
        grid=(grid,),
        in_specs=[
            pl.BlockSpec(memory_space=pltpu.SMEM),
            pl.BlockSpec((1, br, cols), flat),
            pl.BlockSpec((1, br, cols), flat),
        ],
        out_specs=(
            pl.BlockSpec((8, 128), lambda i: (0, 0)),
            pl.BlockSpec((8, 128), lambda i: (0, 0)),
        ),
        out_shape=(
            jax.ShapeDtypeStruct((8, 128), jnp.float32),
            jax.ShapeDtypeStruct((8, 128), jnp.int32),
        ),
    )


def _rank_split(cum, r):
    b = jnp.searchsorted(cum, r, side="right")
    below = jnp.where(b > 0, cum[jnp.maximum(b - 1, 0)], 0)
    return b.astype(jnp.int32), (r - below).astype(jnp.int32)


def kernel(outputs, targets):
    n = outputs.size
    p0 = _build_loss(outputs.shape)
    p1 = _build(n)
    p2 = _build_masked_sum(outputs.shape)
    loss = p0(outputs, targets)

    k_lo = [(n - 1) // 4, (3 * (n - 1)) // 4]
    frac = jnp.array(
        [((n - 1) % 4) / 4.0, ((3 * (n - 1)) % 4) / 4.0], jnp.float32)

    h1 = p1(loss).sum(axis=0)
    c1 = jnp.cumsum(h1)
    b1 = []
    r1 = []
    for k in k_lo:
        b, r = _rank_split(c1, jnp.int32(k))
        b1.append(b)
        r1.append(r)
    b1 = jnp.stack(b1)
    r1 = jnp.stack(r1)

    cnt_b = h1[b1]
    v_start = lax.bitcast_convert_type(b1 << _S1, jnp.float32)
    v_end = lax.bitcast_convert_type((b1 + 1) << _S1, jnp.float32)
    pos = (r1.astype(jnp.float32) + frac + 0.5) / cnt_b.astype(jnp.float32)
    q = v_start + pos * (v_end - v_start)
    thresh = 2.5 * q[1] - 1.5 * q[0]

    thr_smem = thresh.reshape(1, 1).astype(jnp.float32)
    sums, cnts = p2(thr_smem, outputs, targets)
    total = sums[0, 0]
    count = cnts[0, 0]
    return total / count.astype(jnp.float32)


# speedup vs baseline: 64.4130x; 1.0024x over previous
import functools

import jax
import jax.numpy as jnp
from jax import lax
from jax.experimental import pallas as pl
from jax.experimental.pallas import tpu as pltpu
from jax.experimental.pallas import tpu_sc as plsc

_NC = 2
_NS = 16
_NW = _NC * _NS
_L = 16
_CHUNK = 4096
_UNROLL = 4

_S1 = 15
_B1 = 1 << (31 - _S1)


def _mesh():
    return plsc.VectorSubcoreMesh(core_axis_name="c", subcore_axis_name="s")


def _params():
    return pltpu.CompilerParams(needs_layout_passes=False)


def _wid():
    return lax.axis_index("s") * _NC + lax.axis_index("c")


def _zero_hist(hist, words):
    z = jnp.zeros((_L,), jnp.int32)

    def body(b, carry):
        hist[pl.ds(b * _L, _L)] = z
        return carry

    lax.fori_loop(0, words // _L, body, None)


def _stream(l_hbm, bufs, sem0, sem1, base_w, chunks, body_fn, carry_init):
    sems = (sem0, sem1)

    def start(i, p):
        base = pl.multiple_of(base_w + i * _CHUNK, _CHUNK)
        pltpu.async_copy(l_hbm.at[pl.ds(base, _CHUNK)], bufs[p], sems[p])

    def wait(p):
        pltpu.make_async_copy(
            l_hbm.at[pl.ds(0, _CHUNK)], bufs[p], sems[p]).wait()

    start(0, 0)

    def pair(ip, carry):
        i = ip * 2
        start(i + 1, 1)
        wait(0)
        carry = body_fn(bufs[0], carry)

        @pl.when(i + 2 < chunks)
        def _():
            start(i + 2, 0)

        wait(1)
        carry = body_fn(bufs[1], carry)
        return carry

    return lax.fori_loop(0, chunks // 2, pair, carry_init)


def _inner(body1, carry_init):
    def body(jj, carry):
        for v in range(_UNROLL):
            carry = body1(jj * _UNROLL + v, carry)
        return carry

    return lax.fori_loop(0, _CHUNK // _L // _UNROLL, body, carry_init)


@functools.lru_cache(maxsize=None)
def _build(n):
    nw = n // _NW
    chunks = nw // _CHUNK
    assert nw % _CHUNK == 0 and chunks % 2 == 0

    @functools.partial(
        pl.kernel,
        out_type=jax.ShapeDtypeStruct((_NW, _B1), jnp.int32),
        mesh=_mesh(),
        compiler_params=_params(),
        scratch_types=[
            pltpu.VMEM((_CHUNK,), jnp.float32),
            pltpu.VMEM((_CHUNK,), jnp.float32),
            pltpu.VMEM((_B1,), jnp.int32),
            pltpu.SemaphoreType.DMA,
            pltpu.SemaphoreType.DMA,
        ],
    )
    def pass1(l_hbm, out_hbm, lb0, lb1, hist, sem0, sem1):
        ones = jnp.ones((_L,), jnp.int32)
        _zero_hist(hist, _B1)

        def chunk_body(lb, carry):
            def body1(j, c):
                lv = lb[pl.ds(j * _L, _L)]
                u = lax.bitcast_convert_type(lv, jnp.int32)
                plsc.addupdate_scatter(hist, [u >> _S1], ones)
                return c

            return _inner(body1, carry)

        _stream(l_hbm, (lb0, lb1), sem0, sem1,
                _wid() * nw, chunks, chunk_body, 0)
        pltpu.sync_copy(hist, out_hbm.at[_wid()])

    return pass1


@functools.lru_cache(maxsize=None)
def _build_loss(shape, b_lo, b_hi):
    b, rows, cols = shape
    br = 512
    grid = ((b_hi - b_lo) * rows) // br
    bpb = rows // br
    flat3 = lambda i: (b_lo + i // bpb, i % bpb, 0)

    def body(o_ref, t_ref, l_ref):
        d = o_ref[...] - t_ref[...]
        l_ref[...] = (d * d).reshape(-1)

    return pl.pallas_call(
        body,
        grid=(grid,),
        in_specs=[
            pl.BlockSpec((1, br, cols), flat3),
            pl.BlockSpec((1, br, cols), flat3),
        ],
        out_specs=pl.BlockSpec((br * cols,), lambda i: (i,)),
        out_shape=jax.ShapeDtypeStruct(((b_hi - b_lo) * rows * cols,),
                                       jnp.float32),
    )


@functools.lru_cache(maxsize=None)
def _build_masked_sum(shape):
    b, rows, cols = shape
    br = 512
    grid = (b * rows) // br
    assert rows % br == 0

    def body(thr_ref, o_ref, t_ref, sum_ref, cnt_ref):
        i = pl.program_id(0)

        @pl.when(i == 0)
        def _():
            sum_ref[...] = jnp.zeros_like(sum_ref)
            cnt_ref[...] = jnp.zeros_like(cnt_ref)

        d = o_ref[...] - t_ref[...]
        l = d * d
        keep = l < thr_ref[0, 0]
        sum_ref[...] += jnp.sum(jnp.where(keep, l, 0.0))
        cnt_ref[...] += jnp.sum(keep.astype(jnp.int32))

    bpb = rows // br
    flat = lambda i: (i // bpb, i % bpb, 0)
    return pl.pallas_call(
        body,
        grid=(grid,),
        in_specs=[
            pl.BlockSpec(memory_space=pltpu.SMEM),
            pl.BlockSpec((1, br, cols), flat),
            pl.BlockSpec((1, br, cols), flat),
        ],
        out_specs=(
            pl.BlockSpec((8, 128), lambda i: (0, 0)),
            pl.BlockSpec((8, 128), lambda i: (0, 0)),
        ),
        out_shape=(
            jax.ShapeDtypeStruct((8, 128), jnp.float32),
            jax.ShapeDtypeStruct((8, 128), jnp.int32),
        ),
    )


def _rank_split(cum, r):
    b = jnp.searchsorted(cum, r, side="right")
    below = jnp.where(b > 0, cum[jnp.maximum(b - 1, 0)], 0)
    return b.astype(jnp.int32), (r - below).astype(jnp.int32)


def kernel(outputs, targets):
    n = outputs.size
    nb = outputs.shape[0]
    half = nb // 2
    p0a = _build_loss(outputs.shape, 0, half)
    p0b = _build_loss(outputs.shape, half, nb)
    p1 = _build(n // 2)
    p2 = _build_masked_sum(outputs.shape)
    loss_a = p0a(outputs, targets)
    loss_b = p0b(outputs, targets)

    k_lo = [(n - 1) // 4, (3 * (n - 1)) // 4]
    frac = jnp.array(
        [((n - 1) % 4) / 4.0, ((3 * (n - 1)) % 4) / 4.0], jnp.float32)

    h1 = (p1(loss_a) + p1(loss_b)).sum(axis=0)
    c1 = jnp.cumsum(h1)
    b1 = []
    r1 = []
    for k in k_lo:
        b, r = _rank_split(c1, jnp.int32(k))
        b1.append(b)
        r1.append(r)
    b1 = jnp.stack(b1)
    r1 = jnp.stack(r1)

    cnt_b = h1[b1]
    v_start = lax.bitcast_convert_type(b1 << _S1, jnp.float32)
    v_end = lax.bitcast_convert_type((b1 + 1) << _S1, jnp.float32)
    pos = (r1.astype(jnp.float32) + frac + 0.5) / cnt_b.astype(jnp.float32)
    q = v_start + pos * (v_end - v_start)
    thresh = 2.5 * q[1] - 1.5 * q[0]

    thr_smem = thresh.reshape(1, 1).astype(jnp.float32)
    sums, cnts = p2(thr_smem, outputs, targets)
    total = sums[0, 0]
    count = cnts[0, 0]
    return total / count.astype(jnp.float32)
